# baseline (speedup 1.0000x reference)
import jax
import jax.numpy as jnp
from jax import lax
from jax.experimental import pallas as pl
from jax.experimental.pallas import tpu as pltpu
from jax.experimental.pallas import tpu_sc as plsc

N_WORDS = 1000000
EMB_DIM = 32
BATCH = 16384
HIST = 50

NC = 2
NS = 16
NW = NC * NS

BW = BATCH // NW
LANES = 16
ET = EMB_DIM // 8
CC = BW // 128
OUT_FLAT = HIST * ET * 128 * 8 * 128


def _emb_body(xt_hbm, table_hbm, out_hbm, idx_v, buf0, buf1, tb0, tb1,
              gs0, gs1, ss0, ss1):
    bufs = (buf0, buf1)
    tbufs = (tb0, tb1)
    gsems = (gs0, gs1)
    ssems = (ss0, ss1)

    wid = lax.axis_index("s") * NC + lax.axis_index("c")
    pltpu.sync_copy(xt_hbm.at[:, pl.ds(wid * BW, BW)], idx_v)

    def fire_gather(h, s):
        pltpu.async_copy(table_hbm.at[idx_v.at[h]], bufs[s], gsems[s])

    def wait_gather(s):
        pltpu.make_async_copy(table_hbm.at[idx_v.at[0]], bufs[s],
                              gsems[s]).wait()

    def out_off(h, r):
        return ((h * ET + r) * 128 + CC * wid) * 1024

    def fire_store(h, s):
        for r in range(ET):
            pltpu.async_copy(tbufs[s].at[pl.ds(r * 4096, 4096)],
                             out_hbm.at[pl.ds(out_off(h, r), 4096)],
                             ssems[s])

    def wait_store(s):
        for r in range(ET):
            pltpu.make_async_copy(tbufs[s].at[pl.ds(r * 4096, 4096)],
                                  out_hbm.at[pl.ds(0, 4096)],
                                  ssems[s]).wait()

    iota = lax.iota(jnp.int32, LANES)

    def transpose(s):
        buf, tbuf = bufs[s], tbufs[s]

        def q_body(q, carry):
            r = q // 32
            cc = (q // 8) % 4
            ep = q % 8
            col = jnp.full((LANES,), 8 * r + ep, jnp.int32)
            rowbase = cc * 128
            for bg in range(8):
                rows = rowbase + bg * LANES + iota
                v = plsc.load_gather(buf, [rows, col])
                tbuf[pl.ds(q * 128 + bg * LANES, LANES)] = v
            return carry

        lax.fori_loop(0, ET * 4 * 8, q_body, 0)

    fire_gather(0, 0)

    def outer(i, carry):
        for k in range(2):
            h = 2 * i + k

            @pl.when(h + 1 < HIST)
            def _():
                fire_gather(h + 1, 1 - k)

            wait_gather(k)

            @pl.when(h >= 2)
            def _():
                wait_store(k)

            transpose(k)
            fire_store(h, k)
        return carry

    lax.fori_loop(0, HIST // 2, outer, 0)
    wait_store(0)
    wait_store(1)


@jax.jit
def _emb_lookup(xt, table):
    mesh = plsc.VectorSubcoreMesh(core_axis_name="c", subcore_axis_name="s")
    return pl.kernel(
        _emb_body,
        out_type=jax.ShapeDtypeStruct((OUT_FLAT,), jnp.float32),
        mesh=mesh,
        scratch_types=[
            pltpu.VMEM((HIST, BW), jnp.int32),
            pltpu.VMEM((BW, EMB_DIM), jnp.float32),
            pltpu.VMEM((BW, EMB_DIM), jnp.float32),
            pltpu.VMEM((BW * EMB_DIM,), jnp.float32),
            pltpu.VMEM((BW * EMB_DIM,), jnp.float32),
            pltpu.SemaphoreType.DMA,
            pltpu.SemaphoreType.DMA,
            pltpu.SemaphoreType.DMA,
            pltpu.SemaphoreType.DMA,
        ],
        compiler_params=pltpu.CompilerParams(use_tc_tiling_on_sc=False,
                                             needs_layout_passes=False),
    )(xt, table)


def kernel(x, table):
    xt = x.astype(jnp.int32).T
    flat = _emb_lookup(xt, table)
    out = flat.reshape(HIST, ET, 128, 8, 128)
    return out.transpose(2, 4, 0, 1, 3).reshape(BATCH, HIST, EMB_DIM)
